# Initial kernel scaffold; baseline (speedup 1.0000x reference)
import jax
import jax.numpy as jnp
from jax.experimental import pallas as pl


def kernel(h, edge_index, rel_type, norm, weight):
    raise NotImplementedError("write your pallas kernel here")


# speedup vs baseline: 10.6760x; 10.6760x over previous
import functools

import jax
import jax.numpy as jnp
from jax import lax
from jax.experimental import pallas as pl
from jax.experimental.pallas import tpu as pltpu
from jax.experimental.pallas import tpu_sc as plsc

N = 10000
E = 320000
F = 128
R = 8

NC = 2
NS = 16
NW = NC * NS
EW = E // NW
C = 80
K = EW // C
NP = 10240
RPT = NP // NS
ZB = 128


def _matmul_body(h_ref, w_ref, out_ref):
    out_ref[...] = jnp.dot(h_ref[...], w_ref[...],
                           preferred_element_type=jnp.float32)


def _add_body(a_ref, b_ref, out_ref):
    out_ref[...] = a_ref[...] + b_ref[...]


def _edge_body(hw_hbm, src_hbm, rel_hbm, dst_hbm, norm_hbm, out_hbm,
               src_v, rel_v, dst_v, norm_v, idx_v, rows_v, acc, sem):
    cid = lax.axis_index("c")
    sid = lax.axis_index("s")
    wid = sid * NC + cid
    wbase = wid * EW

    zero = jnp.zeros((16,), jnp.float32)

    def zrow(i, _):
        for g in range(F // 16):
            rows_v[i, pl.ds(g * 16, 16)] = zero
        return 0

    lax.fori_loop(0, C, zrow, 0)
    abase = sid * RPT
    for j in range(RPT // C):
        pltpu.sync_copy(rows_v, acc.at[pl.ds(abase + j * C, C)])
    plsc.subcore_barrier()

    def chunk(k, _):
        ebase = wbase + k * C
        pltpu.sync_copy(src_hbm.at[pl.ds(ebase, C)], src_v)
        pltpu.sync_copy(rel_hbm.at[pl.ds(ebase, C)], rel_v)
        pltpu.sync_copy(dst_hbm.at[pl.ds(ebase, C)], dst_v)
        pltpu.sync_copy(norm_hbm.at[pl.ds(ebase, C)], norm_v)
        for j in range(C // 16):
            s = src_v[pl.ds(j * 16, 16)]
            r = rel_v[pl.ds(j * 16, 16)]
            idx_v[pl.ds(j * 16, 16)] = s * R + r
        pltpu.async_copy(hw_hbm.at[idx_v], rows_v, sem).wait()

        def scale(j, _):
            nv = norm_v[pl.ds(j * 16, 16)]
            for i in range(16):
                e = j * 16 + i
                s = nv[i]
                for g in range(F // 16):
                    rows_v[e, pl.ds(g * 16, 16)] = (
                        rows_v[e, pl.ds(g * 16, 16)] * s)
            return 0

        lax.fori_loop(0, C // 16, scale, 0)
        pltpu.sync_copy(rows_v, acc.at[dst_v], add=True)
        return 0

    lax.fori_loop(0, K, chunk, 0)
    plsc.subcore_barrier()

    obase = cid * NP + abase
    pltpu.sync_copy(acc.at[pl.ds(abase, RPT)], out_hbm.at[pl.ds(obase, RPT)])


_edge_kernel = functools.partial(
    pl.kernel,
    out_type=jax.ShapeDtypeStruct((NC * NP, F), jnp.float32),
    mesh=plsc.VectorSubcoreMesh(core_axis_name="c", subcore_axis_name="s"),
    scratch_types=[
        pltpu.VMEM((C,), jnp.int32),
        pltpu.VMEM((C,), jnp.int32),
        pltpu.VMEM((C,), jnp.int32),
        pltpu.VMEM((C,), jnp.float32),
        pltpu.VMEM((C,), jnp.int32),
        pltpu.VMEM((C, F), jnp.float32),
        pltpu.VMEM_SHARED((NP, F), jnp.float32),
        pltpu.SemaphoreType.DMA,
    ],
)(_edge_body)


def kernel(h, edge_index, rel_type, norm, weight):
    wc = weight.transpose(1, 0, 2).reshape(F, R * F)
    hw = pl.pallas_call(
        _matmul_body,
        grid=(25,),
        in_specs=[
            pl.BlockSpec((400, F), lambda i: (i, 0)),
            pl.BlockSpec((F, R * F), lambda i: (0, 0)),
        ],
        out_specs=pl.BlockSpec((400, R * F), lambda i: (i, 0)),
        out_shape=jax.ShapeDtypeStruct((N, R * F), jnp.float32),
    )(h, wc)
    hw = hw.reshape(N * R, F)

    src = edge_index[0]
    rel = rel_type
    dst = edge_index[1]
    nrm = norm.reshape(E)

    partial = _edge_kernel(hw, src, rel, dst, nrm)

    BS = 80
    out = pl.pallas_call(
        _add_body,
        grid=(N // BS,),
        in_specs=[
            pl.BlockSpec((BS, F), lambda i: (i, 0)),
            pl.BlockSpec((BS, F), lambda i: (i + NP // BS, 0)),
        ],
        out_specs=pl.BlockSpec((BS, F), lambda i: (i, 0)),
        out_shape=jax.ShapeDtypeStruct((N, F), jnp.float32),
    )(partial, partial)
    return out


# speedup vs baseline: 11.0890x; 1.0387x over previous
import functools

import jax
import jax.numpy as jnp
from jax import lax
from jax.experimental import pallas as pl
from jax.experimental.pallas import tpu as pltpu
from jax.experimental.pallas import tpu_sc as plsc

N = 10000
E = 320000
F = 128
R = 8

NC = 2
NS = 16
NW = NC * NS
EW = E // NW
C = 80
K = EW // C
NP = 10240
RPT = NP // NS
ZB = 128


def _matmul_body(h_ref, w_ref, out_ref):
    out_ref[...] = jnp.dot(h_ref[...], w_ref[...],
                           preferred_element_type=jnp.float32)


def _add_body(a_ref, b_ref, out_ref):
    out_ref[...] = a_ref[...] + b_ref[...]


def _edge_body(hw_hbm, src_hbm, rel_hbm, dst_hbm, norm_hbm, out_hbm,
               srcb, relb, dstb, nrmb, idxv, rows, acc, isem, gsem):
    cid = lax.axis_index("c")
    sid = lax.axis_index("s")
    wid = sid * NC + cid
    wbase = wid * EW

    zero = jnp.zeros((16,), jnp.float32)

    def zrow(i, _):
        for g in range(F // 16):
            rows[0, i, pl.ds(g * 16, 16)] = zero
        return 0

    lax.fori_loop(0, C, zrow, 0)
    abase = sid * RPT
    for j in range(RPT // C):
        pltpu.sync_copy(rows.at[0], acc.at[pl.ds(abase + j * C, C)])
    plsc.subcore_barrier()

    def fire_idx(c, slot, sp):
        base = wbase + c * C
        pltpu.async_copy(src_hbm.at[pl.ds(base, C)], srcb.at[slot], isem.at[sp])
        pltpu.async_copy(rel_hbm.at[pl.ds(base, C)], relb.at[slot], isem.at[sp])
        pltpu.async_copy(dst_hbm.at[pl.ds(base, C)], dstb.at[slot], isem.at[sp])
        pltpu.async_copy(norm_hbm.at[pl.ds(base, C)], nrmb.at[slot], isem.at[sp])

    def wait_idx(c, slot, sp):
        base = wbase + c * C
        pltpu.make_async_copy(src_hbm.at[pl.ds(base, C)], srcb.at[slot],
                              isem.at[sp]).wait()
        pltpu.make_async_copy(rel_hbm.at[pl.ds(base, C)], relb.at[slot],
                              isem.at[sp]).wait()
        pltpu.make_async_copy(dst_hbm.at[pl.ds(base, C)], dstb.at[slot],
                              isem.at[sp]).wait()
        pltpu.make_async_copy(norm_hbm.at[pl.ds(base, C)], nrmb.at[slot],
                              isem.at[sp]).wait()

    def compute_idx(slot, p):
        for j in range(C // 16):
            s = srcb[slot, pl.ds(j * 16, 16)]
            r = relb[slot, pl.ds(j * 16, 16)]
            idxv[p, pl.ds(j * 16, 16)] = s * R + r

    def fire_gather(p):
        pltpu.async_copy(hw_hbm.at[idxv.at[p]], rows.at[p], gsem.at[p])

    def wait_gather(p):
        pltpu.make_async_copy(hw_hbm.at[idxv.at[p]], rows.at[p],
                              gsem.at[p]).wait()

    fire_idx(0, 0, 0)
    fire_idx(1, 1, 1)
    wait_idx(0, 0, 0)
    compute_idx(0, 0)
    fire_gather(0)

    def chunk(k, _):
        p = lax.rem(k, 2)
        q = 1 - p
        m = lax.rem(k, 4)
        m1 = lax.rem(k + 1, 4)
        m2 = lax.rem(k + 2, 4)

        @pl.when(k + 2 < K)
        def _():
            fire_idx(k + 2, m2, p)

        @pl.when(k + 1 < K)
        def _():
            wait_idx(k + 1, m1, q)
            compute_idx(m1, q)
            fire_gather(q)

        wait_gather(p)

        def scale(j, _):
            nv = nrmb[m, pl.ds(j * 16, 16)]
            for i in range(16):
                e = j * 16 + i
                s = nv[i]
                for g in range(F // 16):
                    rows[p, e, pl.ds(g * 16, 16)] = (
                        rows[p, e, pl.ds(g * 16, 16)] * s)
            return 0

        lax.fori_loop(0, C // 16, scale, 0)
        pltpu.sync_copy(rows.at[p], acc.at[dstb.at[m]], add=True)
        return 0

    lax.fori_loop(0, K, chunk, 0)
    plsc.subcore_barrier()

    obase = cid * NP + abase
    pltpu.sync_copy(acc.at[pl.ds(abase, RPT)], out_hbm.at[pl.ds(obase, RPT)])


_edge_kernel = functools.partial(
    pl.kernel,
    out_type=jax.ShapeDtypeStruct((NC * NP, F), jnp.float32),
    mesh=plsc.VectorSubcoreMesh(core_axis_name="c", subcore_axis_name="s"),
    scratch_types=[
        pltpu.VMEM((4, C), jnp.int32),
        pltpu.VMEM((4, C), jnp.int32),
        pltpu.VMEM((4, C), jnp.int32),
        pltpu.VMEM((4, C), jnp.float32),
        pltpu.VMEM((2, C), jnp.int32),
        pltpu.VMEM((2, C, F), jnp.float32),
        pltpu.VMEM_SHARED((NP, F), jnp.float32),
        pltpu.SemaphoreType.DMA((2,)),
        pltpu.SemaphoreType.DMA((2,)),
    ],
)(_edge_body)


def kernel(h, edge_index, rel_type, norm, weight):
    wc = weight.transpose(1, 0, 2).reshape(F, R * F)
    hw = pl.pallas_call(
        _matmul_body,
        grid=(25,),
        in_specs=[
            pl.BlockSpec((400, F), lambda i: (i, 0)),
            pl.BlockSpec((F, R * F), lambda i: (0, 0)),
        ],
        out_specs=pl.BlockSpec((400, R * F), lambda i: (i, 0)),
        out_shape=jax.ShapeDtypeStruct((N, R * F), jnp.float32),
    )(h, wc)
    hw = hw.reshape(N * R, F)

    src = edge_index[0]
    rel = rel_type
    dst = edge_index[1]
    nrm = norm.reshape(E)

    partial = _edge_kernel(hw, src, rel, dst, nrm)

    BS = 80
    out = pl.pallas_call(
        _add_body,
        grid=(N // BS,),
        in_specs=[
            pl.BlockSpec((BS, F), lambda i: (i, 0)),
            pl.BlockSpec((BS, F), lambda i: (i + NP // BS, 0)),
        ],
        out_specs=pl.BlockSpec((BS, F), lambda i: (i, 0)),
        out_shape=jax.ShapeDtypeStruct((N, F), jnp.float32),
    )(partial, partial)
    return out


# speedup vs baseline: 21.5886x; 1.9469x over previous
import functools

import jax
import jax.numpy as jnp
from jax import lax
from jax.experimental import pallas as pl
from jax.experimental.pallas import tpu as pltpu
from jax.experimental.pallas import tpu_sc as plsc

N = 10000
E = 320000
F = 128
R = 8

NC = 2
NS = 16
NW = NC * NS
EW = E // NW
C = 80
K = EW // C
NP = 10240
RPT = NP // NS
ZB = 128


def _matmul_body(h_ref, w_ref, out_ref):
    out_ref[...] = jnp.dot(h_ref[...], w_ref[...],
                           preferred_element_type=jnp.float32)


def _add_body(a_ref, b_ref, out_ref):
    out_ref[...] = a_ref[...] + b_ref[...]


def _edge_body(hw_hbm, src_hbm, rel_hbm, dst_hbm, norm_hbm, out_hbm,
               srcb, relb, dstb, nrmb, idxv, rows, acc, isem, gsem):
    cid = lax.axis_index("c")
    sid = lax.axis_index("s")
    wid = sid * NC + cid
    wbase = wid * EW

    zero = jnp.zeros((16,), jnp.float32)

    def zrow(i, _):
        for g in range(F // 16):
            rows[0, i, pl.ds(g * 16, 16)] = zero
        return 0

    lax.fori_loop(0, C, zrow, 0)
    abase = sid * RPT
    for j in range(RPT // C):
        pltpu.sync_copy(rows.at[0], acc.at[pl.ds(abase + j * C, C)])
    plsc.subcore_barrier()

    def fire_idx(c, slot, sp):
        base = wbase + c * C
        pltpu.async_copy(src_hbm.at[pl.ds(base, C)], srcb.at[slot], isem.at[sp])
        pltpu.async_copy(rel_hbm.at[pl.ds(base, C)], relb.at[slot], isem.at[sp])
        pltpu.async_copy(dst_hbm.at[pl.ds(base, C)], dstb.at[slot], isem.at[sp])
        pltpu.async_copy(norm_hbm.at[pl.ds(base, C)], nrmb.at[slot], isem.at[sp])

    def wait_idx(c, slot, sp):
        base = wbase + c * C
        pltpu.make_async_copy(src_hbm.at[pl.ds(base, C)], srcb.at[slot],
                              isem.at[sp]).wait()
        pltpu.make_async_copy(rel_hbm.at[pl.ds(base, C)], relb.at[slot],
                              isem.at[sp]).wait()
        pltpu.make_async_copy(dst_hbm.at[pl.ds(base, C)], dstb.at[slot],
                              isem.at[sp]).wait()
        pltpu.make_async_copy(norm_hbm.at[pl.ds(base, C)], nrmb.at[slot],
                              isem.at[sp]).wait()

    def compute_idx(slot, p):
        for j in range(C // 16):
            s = srcb[slot, pl.ds(j * 16, 16)]
            r = relb[slot, pl.ds(j * 16, 16)]
            idxv[p, pl.ds(j * 16, 16)] = s * R + r

    def fire_gather(p):
        pltpu.async_copy(hw_hbm.at[idxv.at[p]], rows.at[p], gsem.at[p])

    def wait_gather(p):
        pltpu.make_async_copy(hw_hbm.at[idxv.at[p]], rows.at[p],
                              gsem.at[p]).wait()

    fire_idx(0, 0, 0)
    fire_idx(1, 1, 1)
    wait_idx(0, 0, 0)
    compute_idx(0, 0)
    fire_gather(0)

    def chunk(k, _):
        p = lax.rem(k, 2)
        q = 1 - p
        m = lax.rem(k, 4)
        m1 = lax.rem(k + 1, 4)
        m2 = lax.rem(k + 2, 4)

        @pl.when(k + 2 < K)
        def _():
            fire_idx(k + 2, m2, p)

        @pl.when(k + 1 < K)
        def _():
            wait_idx(k + 1, m1, q)
            compute_idx(m1, q)
            fire_gather(q)

        wait_gather(p)

        for j in range(C // 16):
            nv = nrmb[m, pl.ds(j * 16, 16)]
            for i in range(16):
                e = j * 16 + i
                s = nv[i]
                for g in range(F // 16):
                    rows[p, e, pl.ds(g * 16, 16)] = (
                        rows[p, e, pl.ds(g * 16, 16)] * s)
        pltpu.sync_copy(rows.at[p], acc.at[dstb.at[m]], add=True)
        return 0

    lax.fori_loop(0, K, chunk, 0)
    plsc.subcore_barrier()

    obase = cid * NP + abase
    pltpu.sync_copy(acc.at[pl.ds(abase, RPT)], out_hbm.at[pl.ds(obase, RPT)])


_edge_kernel = functools.partial(
    pl.kernel,
    out_type=jax.ShapeDtypeStruct((NC * NP, F), jnp.float32),
    mesh=plsc.VectorSubcoreMesh(core_axis_name="c", subcore_axis_name="s"),
    scratch_types=[
        pltpu.VMEM((4, C), jnp.int32),
        pltpu.VMEM((4, C), jnp.int32),
        pltpu.VMEM((4, C), jnp.int32),
        pltpu.VMEM((4, C), jnp.float32),
        pltpu.VMEM((2, C), jnp.int32),
        pltpu.VMEM((2, C, F), jnp.float32),
        pltpu.VMEM_SHARED((NP, F), jnp.float32),
        pltpu.SemaphoreType.DMA((2,)),
        pltpu.SemaphoreType.DMA((2,)),
    ],
)(_edge_body)


def kernel(h, edge_index, rel_type, norm, weight):
    wc = weight.transpose(1, 0, 2).reshape(F, R * F)
    hw = pl.pallas_call(
        _matmul_body,
        grid=(25,),
        in_specs=[
            pl.BlockSpec((400, F), lambda i: (i, 0)),
            pl.BlockSpec((F, R * F), lambda i: (0, 0)),
        ],
        out_specs=pl.BlockSpec((400, R * F), lambda i: (i, 0)),
        out_shape=jax.ShapeDtypeStruct((N, R * F), jnp.float32),
    )(h, wc)
    hw = hw.reshape(N * R, F)

    src = edge_index[0]
    rel = rel_type
    dst = edge_index[1]
    nrm = norm.reshape(E)

    partial = _edge_kernel(hw, src, rel, dst, nrm)

    BS = 80
    out = pl.pallas_call(
        _add_body,
        grid=(N // BS,),
        in_specs=[
            pl.BlockSpec((BS, F), lambda i: (i, 0)),
            pl.BlockSpec((BS, F), lambda i: (i + NP // BS, 0)),
        ],
        out_specs=pl.BlockSpec((BS, F), lambda i: (i, 0)),
        out_shape=jax.ShapeDtypeStruct((N, F), jnp.float32),
    )(partial, partial)
    return out
